# baseline (speedup 1.0000x reference)
import functools

import jax
import jax.numpy as jnp
from jax import lax
from jax.experimental import pallas as pl
from jax.experimental.pallas import tpu as pltpu
from jax.experimental.pallas import tpu_sc as plsc

F_BIN = 13
F_MUL = 13
C_MUL = 8
B = 4096
D_EMB = 1536
D_MODEL = 128

NC, NS = 2, 16
NW = NC * NS
TROWS_B = 2 * F_BIN
TROWS_M = F_MUL * C_MUL
HROWS = F_BIN * B
ROWS = 2 * HROWS
RPW = HROWS // NW
CH = 208
NCH = RPW // CH
NBUF = 2
L = 16


def _bin_table_body(wb_ref, embb_ref, bb_ref, tbin_ref):
    eb = embb_ref[...]
    for c in range(2):
        r = lax.dot_general(eb, wb_ref[c], (((1,), (1,)), ((), ())),
                            preferred_element_type=jnp.float32)
        tbin_ref[pl.ds(13 * c, 13), :] = r + bb_ref[pl.ds(c, 1), :]


def _build_bin_table(Wb, emb_bin, bb):
    return pl.pallas_call(
        _bin_table_body,
        out_shape=jax.ShapeDtypeStruct((TROWS_B, D_MODEL), jnp.float32),
    )(Wb, emb_bin, bb)


def _mul_table_body(wm_ref, embm_ref, bm_ref, tmul_ref):
    w = wm_ref[0]
    v = embm_ref[0]
    t = lax.dot_general(v, w, (((1,), (1,)), ((), ())),
                        preferred_element_type=jnp.float32)
    tmul_ref[0] = t + bm_ref[0]


def _build_mul_table(WmR, embmR, bmR):
    HB = C_MUL * D_MODEL // 4
    return pl.pallas_call(
        _mul_table_body,
        grid=(F_MUL, 4),
        in_specs=[
            pl.BlockSpec((1, HB, D_EMB), lambda f, h: (f, h, 0)),
            pl.BlockSpec((1, 1, D_EMB), lambda f, h: (f, 0, 0)),
            pl.BlockSpec((1, 1, HB), lambda f, h: (f, 0, h)),
        ],
        out_specs=pl.BlockSpec((1, 1, HB), lambda f, h: (f, 0, h)),
        out_shape=jax.ShapeDtypeStruct((F_MUL, 1, C_MUL * D_MODEL),
                                       jnp.float32),
    )(WmR, embmR, bmR)


def _gather_body(row_off, table_hbm, idx_hbm, out_hbm, table_v, idx_v,
                 rows0, rows1, so0, so1):
    wid = lax.axis_index("s") * NC + lax.axis_index("c")
    base = row_off + wid * RPW
    rows = (rows0, rows1)
    so = (so0, so1)
    pltpu.sync_copy(table_hbm, table_v)
    pltpu.sync_copy(idx_hbm.at[pl.ds(wid * RPW, RPW)], idx_v)

    def out_at(j):
        return out_hbm.at[pl.ds((base + j * CH) * D_MODEL, CH * D_MODEL)]

    def compute_chunk(j, buf):
        @plsc.parallel_loop(0, CH // L)
        def group(g):
            rids = idx_v[pl.ds(j * CH + g * L, L)]
            for i in range(L):
                b0 = pl.multiple_of(rids[i] * D_MODEL, L)
                r = g * L + i
                for k in range(D_MODEL // L):
                    buf[pl.ds(r * D_MODEL + k * L, L)] = (
                        table_v[pl.ds(b0 + k * L, L)])

    def body(g, _):
        for s in range(NBUF):
            j = NBUF * g + s
            @pl.when(g > 0)
            def _():
                pltpu.make_async_copy(rows[s], out_at(j), so[s]).wait()
            compute_chunk(j, rows[s])
            pltpu.async_copy(rows[s], out_at(j), so[s])
        return 0

    lax.fori_loop(0, NCH // NBUF, body, 0)
    for s in range(NBUF):
        pltpu.make_async_copy(rows[s], out_at(NCH - NBUF + s), so[s]).wait()


def _gather_call(row_off, trows, out_type):
    return pl.kernel(
        functools.partial(_gather_body, row_off),
        out_type=out_type,
        mesh=plsc.VectorSubcoreMesh(core_axis_name="c", subcore_axis_name="s",
                                    num_cores=NC, num_subcores=NS),
        compiler_params=pltpu.CompilerParams(needs_layout_passes=False),
        scratch_types=[
            pltpu.VMEM((trows * D_MODEL,), jnp.float32),
            pltpu.VMEM((RPW,), jnp.int32),
            pltpu.VMEM((CH * D_MODEL,), jnp.float32),
            pltpu.VMEM((CH * D_MODEL,), jnp.float32),
            pltpu.SemaphoreType.DMA,
            pltpu.SemaphoreType.DMA,
        ],
    )


@functools.lru_cache(maxsize=1)
def _make_gather_bin():
    return _gather_call(0, TROWS_B,
                        jax.ShapeDtypeStruct((ROWS * D_MODEL,), jnp.float32))


@functools.lru_cache(maxsize=1)
def _make_gather_mul():
    return _gather_call(HROWS, TROWS_M, ())


def kernel(x_bin, x_mul, mask, emb_bin, emb_mul, Wb, bb, Wm, bm):
    WmR = Wm.reshape(F_MUL, C_MUL * D_MODEL, D_EMB)
    bmR = bm.reshape(F_MUL, 1, C_MUL * D_MODEL)
    embmR = emb_mul.reshape(F_MUL, 1, D_EMB)
    tbin = _build_bin_table(Wb, emb_bin, bb)
    tmul = _build_mul_table(WmR, embmR, bmR).reshape(-1)

    f_ids = jnp.arange(F_BIN, dtype=jnp.int32)[:, None]
    idx_bin = (x_bin * F_BIN + f_ids).reshape(-1)
    idx_mul = (C_MUL * f_ids + x_mul).reshape(-1)

    out = _make_gather_bin()(tbin.reshape(-1), idx_bin)
    out_ref = jax.new_ref(out)
    _make_gather_mul()(tmul, idx_mul, out_ref)
    return out_ref[...].reshape(F_BIN + F_MUL, B, D_MODEL)


# speedup vs baseline: 1.2031x; 1.2031x over previous
import functools

import jax
import jax.numpy as jnp
from jax import lax
from jax.experimental import pallas as pl
from jax.experimental.pallas import tpu as pltpu
from jax.experimental.pallas import tpu_sc as plsc

F_BIN = 13
F_MUL = 13
C_MUL = 8
B = 4096
D_EMB = 1536
D_MODEL = 128

NC, NS = 2, 16
NW = NC * NS
TROWS_B = 2 * F_BIN
TROWS_M = F_MUL * C_MUL
HROWS = F_BIN * B
ROWS = 2 * HROWS
RPW = HROWS // NW
CH = 208
NCH = RPW // CH
NBUF = 2
L = 16


def _bin_table_body(wb_ref, embb_ref, bb_ref, tbin_ref):
    eb = embb_ref[...]
    for c in range(2):
        r = lax.dot_general(eb, wb_ref[c], (((1,), (1,)), ((), ())),
                            preferred_element_type=jnp.float32)
        tbin_ref[pl.ds(13 * c, 13), :] = r + bb_ref[pl.ds(c, 1), :]


def _build_bin_table(Wb, emb_bin, bb):
    return pl.pallas_call(
        _bin_table_body,
        out_shape=jax.ShapeDtypeStruct((TROWS_B, D_MODEL), jnp.float32),
    )(Wb, emb_bin, bb)


def _mul_table_body(wm_ref, embm_ref, bm_ref, tmul_ref):
    w = wm_ref[0]
    v = embm_ref[0]
    t = lax.dot_general(v, w, (((1,), (1,)), ((), ())),
                        preferred_element_type=jnp.float32)
    tmul_ref[0] = t + bm_ref[0]


def _build_mul_table(WmR, embmR, bmR):
    return pl.pallas_call(
        _mul_table_body,
        grid=(F_MUL,),
        in_specs=[
            pl.BlockSpec((1, C_MUL * D_MODEL, D_EMB), lambda f: (f, 0, 0)),
            pl.BlockSpec((1, 1, D_EMB), lambda f: (f, 0, 0)),
            pl.BlockSpec((1, 1, C_MUL * D_MODEL), lambda f: (f, 0, 0)),
        ],
        out_specs=pl.BlockSpec((1, 1, C_MUL * D_MODEL), lambda f: (f, 0, 0)),
        out_shape=jax.ShapeDtypeStruct((F_MUL, 1, C_MUL * D_MODEL),
                                       jnp.float32),
    )(WmR, embmR, bmR)


def _gather_body(row_off, table_hbm, idx_hbm, out_hbm, table_v, idx_v,
                 rows0, rows1, so0, so1):
    wid = lax.axis_index("s") * NC + lax.axis_index("c")
    base = row_off + wid * RPW
    rows = (rows0, rows1)
    so = (so0, so1)
    pltpu.sync_copy(table_hbm, table_v)
    pltpu.sync_copy(idx_hbm.at[pl.ds(wid * RPW, RPW)], idx_v)

    def out_at(j):
        return out_hbm.at[pl.ds((base + j * CH) * D_MODEL, CH * D_MODEL)]

    def compute_chunk(j, buf):
        @plsc.parallel_loop(0, CH // L)
        def group(g):
            rids = idx_v[pl.ds(j * CH + g * L, L)]
            for i in range(L):
                b0 = pl.multiple_of(rids[i] * D_MODEL, L)
                r = g * L + i
                for k in range(D_MODEL // L):
                    buf[pl.ds(r * D_MODEL + k * L, L)] = (
                        table_v[pl.ds(b0 + k * L, L)])

    def body(g, _):
        for s in range(NBUF):
            j = NBUF * g + s
            @pl.when(g > 0)
            def _():
                pltpu.make_async_copy(rows[s], out_at(j), so[s]).wait()
            compute_chunk(j, rows[s])
            pltpu.async_copy(rows[s], out_at(j), so[s])
        return 0

    lax.fori_loop(0, NCH // NBUF, body, 0)
    for s in range(NBUF):
        pltpu.make_async_copy(rows[s], out_at(NCH - NBUF + s), so[s]).wait()


def _gather_call(row_off, trows, out_type):
    return pl.kernel(
        functools.partial(_gather_body, row_off),
        out_type=out_type,
        mesh=plsc.VectorSubcoreMesh(core_axis_name="c", subcore_axis_name="s",
                                    num_cores=NC, num_subcores=NS),
        compiler_params=pltpu.CompilerParams(needs_layout_passes=False),
        scratch_types=[
            pltpu.VMEM((trows * D_MODEL,), jnp.float32),
            pltpu.VMEM((RPW,), jnp.int32),
            pltpu.VMEM((CH * D_MODEL,), jnp.float32),
            pltpu.VMEM((CH * D_MODEL,), jnp.float32),
            pltpu.SemaphoreType.DMA,
            pltpu.SemaphoreType.DMA,
        ],
    )


@functools.lru_cache(maxsize=1)
def _make_gather_bin():
    return _gather_call(0, TROWS_B,
                        jax.ShapeDtypeStruct((ROWS * D_MODEL,), jnp.float32))


@functools.lru_cache(maxsize=1)
def _make_gather_mul():
    return _gather_call(HROWS, TROWS_M, ())


def kernel(x_bin, x_mul, mask, emb_bin, emb_mul, Wb, bb, Wm, bm):
    WmR = Wm.reshape(F_MUL, C_MUL * D_MODEL, D_EMB)
    bmR = bm.reshape(F_MUL, 1, C_MUL * D_MODEL)
    embmR = emb_mul.reshape(F_MUL, 1, D_EMB)
    tbin = _build_bin_table(Wb, emb_bin, bb)
    tmul = _build_mul_table(WmR, embmR, bmR).reshape(-1)

    f_ids = jnp.arange(F_BIN, dtype=jnp.int32)[:, None]
    idx_bin = (x_bin * F_BIN + f_ids).reshape(-1)
    idx_mul = (C_MUL * f_ids + x_mul).reshape(-1)

    out = _make_gather_bin()(tbin.reshape(-1), idx_bin)
    out_ref = jax.new_ref(out)
    _make_gather_mul()(tmul, idx_mul, out_ref)
    return out_ref[...].reshape(F_BIN + F_MUL, B, D_MODEL)
